# Initial kernel scaffold; baseline (speedup 1.0000x reference)
import jax
import jax.numpy as jnp
from jax.experimental import pallas as pl


def kernel(x, edge_index, edge_feature, W1, b1, W2, b2):
    raise NotImplementedError("write your pallas kernel here")


# speedup vs baseline: 5.7116x; 5.7116x over previous
import functools

import jax
import jax.numpy as jnp
from jax import lax
from jax.experimental import pallas as pl
from jax.experimental.pallas import tpu as pltpu
from jax.experimental.pallas import tpu_sc as plsc

N = 10000
E = 320000
D_IN = 128
D_H = 128
D_OUT = 64
D_EDGE_ = 16

NC = 2
NS = 16
LANES = 16

B = 128
NCHUNK = 79
NE_TILE = B * NCHUNK
E_PAD = NE_TILE * NC * NS
N_PAD = 10240
ROWS_TILE = N_PAD // NS

_MESH = plsc.VectorSubcoreMesh(
    core_axis_name="c", subcore_axis_name="s", num_cores=NC, num_subcores=NS)
_SC_PARAMS = pltpu.CompilerParams(needs_layout_passes=False,
                                  use_tc_tiling_on_sc=False)


def _tc1_body(ef_ref, ew_ref):
    ew_ref[...] = jnp.mean(ef_ref[...], axis=1, keepdims=True)


_EW_GRID = 40
_EW_BLK = E // _EW_GRID


def _tc1_call(edge_feature):
    return pl.pallas_call(
        _tc1_body,
        grid=(_EW_GRID,),
        in_specs=[pl.BlockSpec((_EW_BLK, D_EDGE_), lambda i: (i, 0))],
        out_specs=pl.BlockSpec((_EW_BLK, 1), lambda i: (i, 0)),
        out_shape=jax.ShapeDtypeStruct((E, 1), jnp.float32),
    )(edge_feature)


def _tc2_body(degp_ref, x_ref, w1_ref, dinv_ref, y_ref):
    d = degp_ref[...]
    deg = d[0, :N] + d[1, :N] + 1.0
    dinv = jnp.where(deg > 0, lax.rsqrt(deg), 0.0)[:, None]
    dinv_ref[...] = dinv
    y_ref[...] = dinv * jnp.dot(x_ref[...], w1_ref[...],
                                preferred_element_type=jnp.float32)


def _tc3_body(p0_ref, p1_ref, y1_ref, dinv_ref, b1_ref, w2_ref, y2_ref):
    dinv = dinv_ref[...]
    h = jnp.maximum(dinv * (p0_ref[...] + p1_ref[...] + y1_ref[...])
                    + b1_ref[...], 0.0)
    y2_ref[...] = dinv * jnp.dot(h, w2_ref[...],
                                 preferred_element_type=jnp.float32)


def _tc4_body(q0_ref, q1_ref, y2_ref, dinv_ref, b2_ref, out_ref):
    out_ref[...] = (dinv_ref[...] * (q0_ref[...] + q1_ref[...] + y2_ref[...])
                    + b2_ref[...])


def _tc_call(body, out_shapes, *args):
    return pl.pallas_call(
        body,
        out_shape=[jax.ShapeDtypeStruct(s, jnp.float32) for s in out_shapes],
    )(*args)


def _deg_body(col_hbm, ew_hbm, zero_hbm, out_hbm, idx_v, val_v, acc, sem):
    cid = lax.axis_index("c")
    sid = lax.axis_index("s")
    r0 = sid * ROWS_TILE
    pltpu.sync_copy(zero_hbm.at[pl.ds(r0, ROWS_TILE)],
                    acc.at[pl.ds(r0, ROWS_TILE)])
    plsc.subcore_barrier()

    base = cid * (NE_TILE * NS) + sid * NE_TILE

    def chunk(j, carry):
        off = base + j * B
        pltpu.sync_copy(col_hbm.at[pl.ds(off, B)], idx_v)
        pltpu.sync_copy(ew_hbm.at[pl.ds(off, B)], val_v)
        pltpu.sync_copy(val_v, acc.at[idx_v], add=True)
        return carry

    lax.fori_loop(0, NCHUNK, chunk, 0)
    plsc.subcore_barrier()
    pltpu.sync_copy(acc.at[pl.ds(r0, ROWS_TILE)],
                    out_hbm.at[pl.ds(cid * N_PAD + r0, ROWS_TILE)])


_deg_kernel = functools.partial(
    pl.kernel,
    out_type=jax.ShapeDtypeStruct((NC * N_PAD,), jnp.float32),
    mesh=_MESH,
    scratch_types=[
        pltpu.VMEM((B,), jnp.int32),
        pltpu.VMEM((B,), jnp.float32),
        pltpu.VMEM_SHARED((N_PAD,), jnp.float32),
        pltpu.SemaphoreType.DMA,
    ],
    compiler_params=_SC_PARAMS,
)(_deg_body)


def _make_edge_pass(D):
    def body(y_hbm, row_hbm, col_hbm, ew_hbm, zero_hbm, out_hbm,
             idx_r, idx_c, ewc, rows, acc, sem):
        cvecs = [lax.iota(jnp.int32, LANES) + g * LANES
                 for g in range(D // LANES)]
        cid = lax.axis_index("c")
        sid = lax.axis_index("s")
        r0 = sid * ROWS_TILE
        pltpu.sync_copy(zero_hbm.at[pl.ds(r0, ROWS_TILE)],
                        acc.at[pl.ds(r0, ROWS_TILE)])
        plsc.subcore_barrier()

        base = cid * (NE_TILE * NS) + sid * NE_TILE

        def chunk(j, carry):
            off = base + j * B
            pltpu.sync_copy(row_hbm.at[pl.ds(off, B)], idx_r)
            pltpu.sync_copy(col_hbm.at[pl.ds(off, B)], idx_c)
            pltpu.sync_copy(ew_hbm.at[pl.ds(off, B)], ewc)
            pltpu.async_copy(y_hbm.at[idx_r], rows, sem).wait()

            def scale_one(e, evec):
                sv = plsc.load_gather(ewc, [evec])
                for cv in cvecs:
                    v = plsc.load_gather(rows, [evec, cv])
                    plsc.store_scatter(rows, [evec, cv], v * sv)
                return evec + 1

            lax.fori_loop(0, B, scale_one, jnp.zeros((LANES,), jnp.int32))
            pltpu.sync_copy(rows, acc.at[idx_c], add=True)
            return carry

        lax.fori_loop(0, NCHUNK, chunk, 0)
        plsc.subcore_barrier()
        pltpu.sync_copy(acc.at[pl.ds(r0, ROWS_TILE)],
                        out_hbm.at[pl.ds(cid * N_PAD + r0, ROWS_TILE)])

    return pl.kernel(
        body,
        out_type=jax.ShapeDtypeStruct((NC * N_PAD, D), jnp.float32),
        mesh=_MESH,
        scratch_types=[
            pltpu.VMEM((B,), jnp.int32),
            pltpu.VMEM((B,), jnp.int32),
            pltpu.VMEM((B,), jnp.float32),
            pltpu.VMEM((B, D), jnp.float32),
            pltpu.VMEM_SHARED((N_PAD, D), jnp.float32),
            pltpu.SemaphoreType.DMA,
        ],
        compiler_params=_SC_PARAMS,
    )


_edge_pass_128 = _make_edge_pass(D_H)
_edge_pass_64 = _make_edge_pass(D_OUT)


def kernel(x, edge_index, edge_feature, W1, b1, W2, b2):
    row = edge_index[0].astype(jnp.int32)
    col = edge_index[1].astype(jnp.int32)
    pad = E_PAD - E
    row_p = jnp.concatenate([row, jnp.zeros((pad,), jnp.int32)])
    col_p = jnp.concatenate([col, jnp.zeros((pad,), jnp.int32)])

    ew2d = _tc1_call(edge_feature)
    ew_p = jnp.concatenate([ew2d[:, 0], jnp.zeros((pad,), jnp.float32)])

    zeros1 = jnp.zeros((N_PAD,), jnp.float32)
    degp = _deg_kernel(col_p, ew_p, zeros1).reshape(NC, N_PAD)

    dinv, y1 = _tc_call(_tc2_body, [(N, 1), (N, D_H)], degp, x, W1)

    zeros_h = jnp.zeros((N_PAD, D_H), jnp.float32)
    p = _edge_pass_128(y1, row_p, col_p, ew_p, zeros_h).reshape(NC, N_PAD, D_H)

    y2, = _tc_call(_tc3_body, [(N, D_OUT)],
                   p[0, :N], p[1, :N], y1, dinv, b1.reshape(1, D_H), W2)

    zeros_o = jnp.zeros((N_PAD, D_OUT), jnp.float32)
    q = _edge_pass_64(y2, row_p, col_p, ew_p, zeros_o).reshape(NC, N_PAD, D_OUT)

    out, = _tc_call(_tc4_body, [(N, D_OUT)],
                    q[0, :N], q[1, :N], y2, dinv, b2.reshape(1, D_OUT))
    return (out, out)


# speedup vs baseline: 8.0041x; 1.4014x over previous
import functools

import jax
import jax.numpy as jnp
from jax import lax
from jax.experimental import pallas as pl
from jax.experimental.pallas import tpu as pltpu
from jax.experimental.pallas import tpu_sc as plsc

N = 10000
E = 320000
D_IN = 128
D_H = 128
D_OUT = 64
D_EDGE_ = 16

NC = 2
NS = 16
LANES = 16

B = 128
NCHUNK = 80
NE_TILE = B * NCHUNK
NW = NC * NS
E_PAD = NE_TILE * NW
N_PAD = 10240
ROWS_TILE = N_PAD // NS
NPHASE = 4
CHUNK_P = NCHUNK // NPHASE

_MESH = plsc.VectorSubcoreMesh(
    core_axis_name="c", subcore_axis_name="s", num_cores=NC, num_subcores=NS)
_SC_PARAMS = pltpu.CompilerParams(needs_layout_passes=False,
                                  use_tc_tiling_on_sc=False)


def _tc1_body(ef_ref, ew_ref):
    ew_ref[...] = jnp.mean(ef_ref[...], axis=1, keepdims=True)


_EW_GRID = 40
_EW_BLK = E // _EW_GRID


def _tc1_call(edge_feature):
    return pl.pallas_call(
        _tc1_body,
        grid=(_EW_GRID,),
        in_specs=[pl.BlockSpec((_EW_BLK, D_EDGE_), lambda i: (i, 0))],
        out_specs=pl.BlockSpec((_EW_BLK, 1), lambda i: (i, 0)),
        out_shape=jax.ShapeDtypeStruct((E, 1), jnp.float32),
    )(edge_feature)


def _tc2_body(degp_ref, x_ref, w1_ref, dinv_ref, y_ref):
    d = degp_ref[...]
    deg = d[0, :N] + d[1, :N] + 1.0
    dinv = jnp.where(deg > 0, lax.rsqrt(deg), 0.0)[:, None]
    dinv_ref[...] = dinv
    y_ref[...] = dinv * jnp.dot(x_ref[...], w1_ref[...],
                                preferred_element_type=jnp.float32)


def _tc3_body(p0_ref, p1_ref, y1_ref, dinv_ref, b1_ref, w2_ref, y2_ref):
    dinv = dinv_ref[...]
    h = jnp.maximum(dinv * (p0_ref[...] + p1_ref[...] + y1_ref[...])
                    + b1_ref[...], 0.0)
    y2_ref[...] = dinv * jnp.dot(h, w2_ref[...],
                                 preferred_element_type=jnp.float32)


def _tc4_body(q0_ref, q1_ref, y2_ref, dinv_ref, b2_ref, out_ref):
    out_ref[...] = (dinv_ref[...] * (q0_ref[...] + q1_ref[...] + y2_ref[...])
                    + b2_ref[...])


def _tc_call(body, out_shapes, *args):
    return pl.pallas_call(
        body,
        out_shape=[jax.ShapeDtypeStruct(s, jnp.float32) for s in out_shapes],
    )(*args)


_DEG_WAVE = 16


def _deg_body(col3_hbm, ew3_hbm, zero_hbm, out_hbm, idx_all, val_all, acc, sem):
    cid = lax.axis_index("c")
    sid = lax.axis_index("s")
    w = cid * NS + sid
    r0 = sid * ROWS_TILE
    pltpu.sync_copy(zero_hbm.at[pl.ds(r0, ROWS_TILE)],
                    acc.at[pl.ds(r0, ROWS_TILE)])
    pltpu.sync_copy(col3_hbm.at[w], idx_all)
    pltpu.sync_copy(ew3_hbm.at[w], val_all)
    plsc.subcore_barrier()

    def wave(t, carry):
        for k in range(_DEG_WAVE):
            j = t * _DEG_WAVE + k
            pltpu.async_copy(val_all.at[j], acc.at[idx_all.at[j]], sem,
                             add=True)
        for k in range(_DEG_WAVE):
            pltpu.make_async_copy(val_all.at[0], acc.at[idx_all.at[0]],
                                  sem).wait()
        return carry

    lax.fori_loop(0, NCHUNK // _DEG_WAVE, wave, 0)
    plsc.subcore_barrier()
    pltpu.sync_copy(acc.at[pl.ds(r0, ROWS_TILE)],
                    out_hbm.at[pl.ds(cid * N_PAD + r0, ROWS_TILE)])


_deg_kernel = functools.partial(
    pl.kernel,
    out_type=jax.ShapeDtypeStruct((NC * N_PAD,), jnp.float32),
    mesh=_MESH,
    scratch_types=[
        pltpu.VMEM((NCHUNK, B), jnp.int32),
        pltpu.VMEM((NCHUNK, B), jnp.float32),
        pltpu.VMEM_SHARED((N_PAD,), jnp.float32),
        pltpu.SemaphoreType.DMA,
    ],
    compiler_params=_SC_PARAMS,
)(_deg_body)


def _make_edge_pass(D):
    def body(y_hbm, row3_hbm, col3_hbm, ew3_hbm, zero_hbm, out_hbm,
             row_all, col_all, ew_all, rows0, rows1, acc, g0, g1, s0, s1):
        cvecs = [lax.iota(jnp.int32, LANES) + g * LANES
                 for g in range(D // LANES)]
        cid = lax.axis_index("c")
        sid = lax.axis_index("s")
        w = cid * NS + sid
        r0 = sid * ROWS_TILE
        pltpu.sync_copy(zero_hbm.at[pl.ds(r0, ROWS_TILE)],
                        acc.at[pl.ds(r0, ROWS_TILE)])
        plsc.subcore_barrier()

        def start_gather(j, buf, sem):
            pltpu.async_copy(y_hbm.at[row_all.at[j]], buf, sem)

        def wait_gather(buf, sem):
            pltpu.make_async_copy(y_hbm.at[row_all.at[0]], buf, sem).wait()

        def start_scat(j, buf, sem):
            pltpu.async_copy(buf, acc.at[col_all.at[j]], sem, add=True)

        def wait_scat(buf, sem):
            pltpu.make_async_copy(buf, acc.at[col_all.at[0]], sem).wait()

        def scale(buf, j):
            jvec = jnp.full((LANES,), j, jnp.int32)

            def one(e, evec):
                sv = plsc.load_gather(ew_all, [jvec, evec])
                for cv in cvecs:
                    v = plsc.load_gather(buf, [evec, cv])
                    plsc.store_scatter(buf, [evec, cv], v * sv)
                return evec + 1

            lax.fori_loop(0, B, one, jnp.zeros((LANES,), jnp.int32))

        nloop = CHUNK_P // 2

        def phase(p, carry):
            base = p * CHUNK_P
            d1 = pltpu.async_copy(row3_hbm.at[w, pl.ds(base, CHUNK_P)],
                                  row_all, g0)
            d2 = pltpu.async_copy(col3_hbm.at[w, pl.ds(base, CHUNK_P)],
                                  col_all, g0)
            d3 = pltpu.async_copy(ew3_hbm.at[w, pl.ds(base, CHUNK_P)],
                                  ew_all, g0)
            d1.wait()
            d2.wait()
            d3.wait()

            start_gather(0, rows0, g0)

            def body2(jj, carry2):
                j0 = jj * 2
                j1 = j0 + 1
                wait_gather(rows0, g0)

                @pl.when(jj > 0)
                def _():
                    wait_scat(rows1, s1)

                start_gather(j1, rows1, g1)
                scale(rows0, j0)
                start_scat(j0, rows0, s0)

                wait_gather(rows1, g1)

                @pl.when(jj < nloop - 1)
                def _():
                    wait_scat(rows0, s0)
                    start_gather(j0 + 2, rows0, g0)

                scale(rows1, j1)
                start_scat(j1, rows1, s1)
                return carry2

            lax.fori_loop(0, nloop, body2, 0)
            wait_scat(rows0, s0)
            wait_scat(rows1, s1)
            return carry

        lax.fori_loop(0, NPHASE, phase, 0)
        plsc.subcore_barrier()
        pltpu.sync_copy(acc.at[pl.ds(r0, ROWS_TILE)],
                        out_hbm.at[pl.ds(cid * N_PAD + r0, ROWS_TILE)])

    return pl.kernel(
        body,
        out_type=jax.ShapeDtypeStruct((NC * N_PAD, D), jnp.float32),
        mesh=_MESH,
        scratch_types=[
            pltpu.VMEM((CHUNK_P, B), jnp.int32),
            pltpu.VMEM((CHUNK_P, B), jnp.int32),
            pltpu.VMEM((CHUNK_P, B), jnp.float32),
            pltpu.VMEM((B, D), jnp.float32),
            pltpu.VMEM((B, D), jnp.float32),
            pltpu.VMEM_SHARED((N_PAD, D), jnp.float32),
            pltpu.SemaphoreType.DMA,
            pltpu.SemaphoreType.DMA,
            pltpu.SemaphoreType.DMA,
            pltpu.SemaphoreType.DMA,
        ],
        compiler_params=_SC_PARAMS,
    )


_edge_pass_128 = _make_edge_pass(D_H)
_edge_pass_64 = _make_edge_pass(D_OUT)


def kernel(x, edge_index, edge_feature, W1, b1, W2, b2):
    row = edge_index[0].astype(jnp.int32)
    col = edge_index[1].astype(jnp.int32)
    pad = E_PAD - E
    row_p = jnp.concatenate([row, jnp.zeros((pad,), jnp.int32)]
                            ).reshape(NW, NCHUNK, B)
    col_p = jnp.concatenate([col, jnp.zeros((pad,), jnp.int32)]
                            ).reshape(NW, NCHUNK, B)

    ew2d = _tc1_call(edge_feature)
    ew_p = jnp.concatenate([ew2d[:, 0], jnp.zeros((pad,), jnp.float32)]
                           ).reshape(NW, NCHUNK, B)

    zeros1 = jnp.zeros((N_PAD,), jnp.float32)
    degp = _deg_kernel(col_p, ew_p, zeros1).reshape(NC, N_PAD)

    dinv, y1 = _tc_call(_tc2_body, [(N, 1), (N, D_H)], degp, x, W1)

    zeros_h = jnp.zeros((N_PAD, D_H), jnp.float32)
    p = _edge_pass_128(y1, row_p, col_p, ew_p, zeros_h).reshape(NC, N_PAD, D_H)

    y2, = _tc_call(_tc3_body, [(N, D_OUT)],
                   p[0, :N], p[1, :N], y1, dinv, b1.reshape(1, D_H), W2)

    zeros_o = jnp.zeros((N_PAD, D_OUT), jnp.float32)
    q = _edge_pass_64(y2, row_p, col_p, ew_p, zeros_o).reshape(NC, N_PAD, D_OUT)

    out, = _tc_call(_tc4_body, [(N, D_OUT)],
                    q[0, :N], q[1, :N], y2, dinv, b2.reshape(1, D_OUT))
    return (out, out)


# speedup vs baseline: 9.0246x; 1.1275x over previous
import functools

import jax
import jax.numpy as jnp
from jax import lax
from jax.experimental import pallas as pl
from jax.experimental.pallas import tpu as pltpu
from jax.experimental.pallas import tpu_sc as plsc

N = 10000
E = 320000
D_IN = 128
D_H = 128
D_OUT = 64
D_EDGE_ = 16

NC = 2
NS = 16
LANES = 16

B = 128
NCHUNK = 80
NE_TILE = B * NCHUNK
NW = NC * NS
E_PAD = NE_TILE * NW
N_PAD = 10240
ROWS_TILE = N_PAD // NS
NPHASE = 4
CHUNK_P = NCHUNK // NPHASE

_MESH = plsc.VectorSubcoreMesh(
    core_axis_name="c", subcore_axis_name="s", num_cores=NC, num_subcores=NS)
_SC_PARAMS = pltpu.CompilerParams(needs_layout_passes=False,
                                  use_tc_tiling_on_sc=False)


def _tc1_body(ef_ref, ew_ref):
    ew_ref[...] = jnp.mean(ef_ref[...], axis=1, keepdims=True)


_EW_GRID = 40
_EW_BLK = E // _EW_GRID


def _tc1_call(edge_feature):
    return pl.pallas_call(
        _tc1_body,
        grid=(_EW_GRID,),
        in_specs=[pl.BlockSpec((_EW_BLK, D_EDGE_), lambda i: (i, 0))],
        out_specs=pl.BlockSpec((_EW_BLK, 1), lambda i: (i, 0)),
        out_shape=jax.ShapeDtypeStruct((E, 1), jnp.float32),
    )(edge_feature)


def _tc2_body(degp_ref, x_ref, w1_ref, dinv_ref, y_ref):
    d = degp_ref[...]
    deg = d[0, :N] + d[1, :N] + 1.0
    dinv = jnp.where(deg > 0, lax.rsqrt(deg), 0.0)[:, None]
    dinv_ref[...] = dinv
    y_ref[...] = dinv * jnp.dot(x_ref[...], w1_ref[...],
                                preferred_element_type=jnp.float32)


def _tc3_body(p0_ref, p1_ref, y1_ref, dinv_ref, b1_ref, w2_ref, y2_ref):
    dinv = dinv_ref[...]
    h = jnp.maximum(dinv * (p0_ref[...] + p1_ref[...] + y1_ref[...])
                    + b1_ref[...], 0.0)
    y2_ref[...] = dinv * jnp.dot(h, w2_ref[...],
                                 preferred_element_type=jnp.float32)


def _tc4_body(q0_ref, q1_ref, y2_ref, dinv_ref, b2_ref, out_ref):
    out_ref[...] = (dinv_ref[...] * (q0_ref[...] + q1_ref[...] + y2_ref[...])
                    + b2_ref[...])


def _tc_call(body, out_shapes, *args):
    return pl.pallas_call(
        body,
        out_shape=[jax.ShapeDtypeStruct(s, jnp.float32) for s in out_shapes],
    )(*args)


_DEG_WAVE = 16


def _deg_body(col3_hbm, ew3_hbm, zero_hbm, out_hbm, idx_all, val_all, acc, sem):
    cid = lax.axis_index("c")
    sid = lax.axis_index("s")
    w = cid * NS + sid
    r0 = sid * ROWS_TILE
    pltpu.sync_copy(zero_hbm.at[pl.ds(r0, ROWS_TILE)],
                    acc.at[pl.ds(r0, ROWS_TILE)])
    pltpu.sync_copy(col3_hbm.at[w], idx_all)
    pltpu.sync_copy(ew3_hbm.at[w], val_all)
    plsc.subcore_barrier()

    def wave(t, carry):
        for k in range(_DEG_WAVE):
            j = t * _DEG_WAVE + k
            pltpu.async_copy(val_all.at[j], acc.at[idx_all.at[j]], sem,
                             add=True)
        for k in range(_DEG_WAVE):
            pltpu.make_async_copy(val_all.at[0], acc.at[idx_all.at[0]],
                                  sem).wait()
        return carry

    lax.fori_loop(0, NCHUNK // _DEG_WAVE, wave, 0)
    plsc.subcore_barrier()
    pltpu.sync_copy(acc.at[pl.ds(r0, ROWS_TILE)],
                    out_hbm.at[pl.ds(cid * N_PAD + r0, ROWS_TILE)])


_deg_kernel = functools.partial(
    pl.kernel,
    out_type=jax.ShapeDtypeStruct((NC * N_PAD,), jnp.float32),
    mesh=_MESH,
    scratch_types=[
        pltpu.VMEM((NCHUNK, B), jnp.int32),
        pltpu.VMEM((NCHUNK, B), jnp.float32),
        pltpu.VMEM_SHARED((N_PAD,), jnp.float32),
        pltpu.SemaphoreType.DMA,
    ],
    compiler_params=_SC_PARAMS,
)(_deg_body)


def _make_edge_pass(D):
    def body(y_hbm, row3_hbm, col3_hbm, ew3_hbm, zero_hbm, out_hbm,
             row_all, col_all, ew_all, rows0, rows1, acc, g0, g1, s0, s1):
        cvecs = [lax.iota(jnp.int32, LANES) + g * LANES
                 for g in range(D // LANES)]
        cid = lax.axis_index("c")
        sid = lax.axis_index("s")
        w = cid * NS + sid
        r0 = sid * ROWS_TILE
        pltpu.sync_copy(zero_hbm.at[pl.ds(r0, ROWS_TILE)],
                        acc.at[pl.ds(r0, ROWS_TILE)])
        plsc.subcore_barrier()

        def start_gather(j, buf, sem):
            pltpu.async_copy(y_hbm.at[row_all.at[j]], buf, sem)

        def wait_gather(buf, sem):
            pltpu.make_async_copy(y_hbm.at[row_all.at[0]], buf, sem).wait()

        def start_scat(j, buf, sem):
            pltpu.async_copy(buf, acc.at[col_all.at[j]], sem, add=True)

        def wait_scat(buf, sem):
            pltpu.make_async_copy(buf, acc.at[col_all.at[0]], sem).wait()

        UNROLL = 4

        def scale(buf, j):
            jvec = jnp.full((LANES,), j, jnp.int32)

            def one(i, evec):
                e0 = i * UNROLL
                svs = [plsc.load_gather(ew_all, [jvec, evec + u])
                       for u in range(UNROLL)]
                for u in range(UNROLL):
                    for g in range(D // LANES):
                        sl = pl.ds(g * LANES, LANES)
                        buf[e0 + u, sl] = buf[e0 + u, sl] * svs[u]
                return evec + UNROLL

            lax.fori_loop(0, B // UNROLL, one,
                          jnp.zeros((LANES,), jnp.int32))

        nloop = CHUNK_P // 2

        def phase(p, carry):
            base = p * CHUNK_P
            d1 = pltpu.async_copy(row3_hbm.at[w, pl.ds(base, CHUNK_P)],
                                  row_all, g0)
            d2 = pltpu.async_copy(col3_hbm.at[w, pl.ds(base, CHUNK_P)],
                                  col_all, g0)
            d3 = pltpu.async_copy(ew3_hbm.at[w, pl.ds(base, CHUNK_P)],
                                  ew_all, g0)
            d1.wait()
            d2.wait()
            d3.wait()

            start_gather(0, rows0, g0)

            def body2(jj, carry2):
                j0 = jj * 2
                j1 = j0 + 1
                wait_gather(rows0, g0)

                @pl.when(jj > 0)
                def _():
                    wait_scat(rows1, s1)

                start_gather(j1, rows1, g1)
                scale(rows0, j0)
                start_scat(j0, rows0, s0)

                wait_gather(rows1, g1)

                @pl.when(jj < nloop - 1)
                def _():
                    wait_scat(rows0, s0)
                    start_gather(j0 + 2, rows0, g0)

                scale(rows1, j1)
                start_scat(j1, rows1, s1)
                return carry2

            lax.fori_loop(0, nloop, body2, 0)
            wait_scat(rows0, s0)
            wait_scat(rows1, s1)
            return carry

        lax.fori_loop(0, NPHASE, phase, 0)
        plsc.subcore_barrier()
        pltpu.sync_copy(acc.at[pl.ds(r0, ROWS_TILE)],
                        out_hbm.at[pl.ds(cid * N_PAD + r0, ROWS_TILE)])

    return pl.kernel(
        body,
        out_type=jax.ShapeDtypeStruct((NC * N_PAD, D), jnp.float32),
        mesh=_MESH,
        scratch_types=[
            pltpu.VMEM((CHUNK_P, B), jnp.int32),
            pltpu.VMEM((CHUNK_P, B), jnp.int32),
            pltpu.VMEM((CHUNK_P, B), jnp.float32),
            pltpu.VMEM((B, D), jnp.float32),
            pltpu.VMEM((B, D), jnp.float32),
            pltpu.VMEM_SHARED((N_PAD, D), jnp.float32),
            pltpu.SemaphoreType.DMA,
            pltpu.SemaphoreType.DMA,
            pltpu.SemaphoreType.DMA,
            pltpu.SemaphoreType.DMA,
        ],
        compiler_params=_SC_PARAMS,
    )


_edge_pass_128 = _make_edge_pass(D_H)
_edge_pass_64 = _make_edge_pass(D_OUT)


def kernel(x, edge_index, edge_feature, W1, b1, W2, b2):
    row = edge_index[0].astype(jnp.int32)
    col = edge_index[1].astype(jnp.int32)
    pad = E_PAD - E
    row_p = jnp.concatenate([row, jnp.zeros((pad,), jnp.int32)]
                            ).reshape(NW, NCHUNK, B)
    col_p = jnp.concatenate([col, jnp.zeros((pad,), jnp.int32)]
                            ).reshape(NW, NCHUNK, B)

    ew2d = _tc1_call(edge_feature)
    ew_p = jnp.concatenate([ew2d[:, 0], jnp.zeros((pad,), jnp.float32)]
                           ).reshape(NW, NCHUNK, B)

    zeros1 = jnp.zeros((N_PAD,), jnp.float32)
    degp = _deg_kernel(col_p, ew_p, zeros1).reshape(NC, N_PAD)

    dinv, y1 = _tc_call(_tc2_body, [(N, 1), (N, D_H)], degp, x, W1)

    zeros_h = jnp.zeros((N_PAD, D_H), jnp.float32)
    p = _edge_pass_128(y1, row_p, col_p, ew_p, zeros_h).reshape(NC, N_PAD, D_H)

    y2, = _tc_call(_tc3_body, [(N, D_OUT)],
                   p[0, :N], p[1, :N], y1, dinv, b1.reshape(1, D_H), W2)

    zeros_o = jnp.zeros((N_PAD, D_OUT), jnp.float32)
    q = _edge_pass_64(y2, row_p, col_p, ew_p, zeros_o).reshape(NC, N_PAD, D_OUT)

    out, = _tc_call(_tc4_body, [(N, D_OUT)],
                    q[0, :N], q[1, :N], y2, dinv, b2.reshape(1, D_OUT))
    return (out, out)


# speedup vs baseline: 10.7800x; 1.1945x over previous
import functools

import numpy as np

import jax
import jax.numpy as jnp
from jax import lax
from jax.experimental import pallas as pl
from jax.experimental.pallas import tpu as pltpu
from jax.experimental.pallas import tpu_sc as plsc

N = 10000
E = 320000
D_IN = 128
D_H = 128
D_OUT = 64
D_EDGE_ = 16

NC = 2
NS = 16
LANES = 16

B = 128
NCHUNK = 80
NE_TILE = B * NCHUNK
NW = NC * NS
E_PAD = NE_TILE * NW
N_PAD = 10240
ROWS_TILE = N_PAD // NS
NPHASE = 4
CHUNK_P = NCHUNK // NPHASE


def _bf16_col_order(D):
    perm = np.empty((D,), np.int64)
    for g in range(D // 32):
        for k in range(16):
            perm[32 * g + k] = 32 * g + 2 * k
            perm[32 * g + 16 + k] = 32 * g + 2 * k + 1
    return np.argsort(perm)


_Q128 = _bf16_col_order(D_H)
_Q64 = _bf16_col_order(D_OUT)

_MESH = plsc.VectorSubcoreMesh(
    core_axis_name="c", subcore_axis_name="s", num_cores=NC, num_subcores=NS)
_SC_PARAMS = pltpu.CompilerParams(needs_layout_passes=False,
                                  use_tc_tiling_on_sc=False)


def _tc1_body(ef_ref, ew_ref):
    ew_ref[...] = jnp.mean(ef_ref[...], axis=1, keepdims=True)


_EW_GRID = 40
_EW_BLK = E // _EW_GRID


def _tc1_call(edge_feature):
    return pl.pallas_call(
        _tc1_body,
        grid=(_EW_GRID,),
        in_specs=[pl.BlockSpec((_EW_BLK, D_EDGE_), lambda i: (i, 0))],
        out_specs=pl.BlockSpec((_EW_BLK, 1), lambda i: (i, 0)),
        out_shape=jax.ShapeDtypeStruct((E, 1), jnp.float32),
    )(edge_feature)


def _tc2_body(degp_ref, x_ref, w1_ref, dinv_ref, y_ref):
    d = degp_ref[...]
    deg = d[0, :N] + d[1, :N] + 1.0
    dinv = jnp.where(deg > 0, lax.rsqrt(deg), 0.0)[:, None]
    dinv_ref[...] = dinv
    y_ref[...] = dinv * jnp.dot(x_ref[...], w1_ref[...],
                                preferred_element_type=jnp.float32)


def _tc3_body(p0_ref, p1_ref, y1_ref, dinv_ref, b1_ref, w2_ref, y2_ref):
    dinv = dinv_ref[...]
    h = jnp.maximum(dinv * (p0_ref[...] + p1_ref[...] + y1_ref[...])
                    + b1_ref[...], 0.0)
    y2_ref[...] = dinv * jnp.dot(h, w2_ref[...],
                                 preferred_element_type=jnp.float32)


def _tc4_body(q0_ref, q1_ref, y2_ref, dinv_ref, b2_ref, out_ref):
    out_ref[...] = (dinv_ref[...] * (q0_ref[...] + q1_ref[...] + y2_ref[...])
                    + b2_ref[...])


def _tc_call(body, out_shapes, *args):
    return pl.pallas_call(
        body,
        out_shape=[jax.ShapeDtypeStruct(s, jnp.float32) for s in out_shapes],
    )(*args)


_DEG_WAVE = 16


def _deg_body(col3_hbm, ew3_hbm, zero_hbm, out_hbm, idx_all, val_all, acc, sem):
    cid = lax.axis_index("c")
    sid = lax.axis_index("s")
    w = cid * NS + sid
    r0 = sid * ROWS_TILE
    pltpu.sync_copy(zero_hbm.at[pl.ds(r0, ROWS_TILE)],
                    acc.at[pl.ds(r0, ROWS_TILE)])
    pltpu.sync_copy(col3_hbm.at[w], idx_all)
    pltpu.sync_copy(ew3_hbm.at[w], val_all)
    plsc.subcore_barrier()

    def wave(t, carry):
        for k in range(_DEG_WAVE):
            j = t * _DEG_WAVE + k
            pltpu.async_copy(val_all.at[j], acc.at[idx_all.at[j]], sem,
                             add=True)
        for k in range(_DEG_WAVE):
            pltpu.make_async_copy(val_all.at[0], acc.at[idx_all.at[0]],
                                  sem).wait()
        return carry

    lax.fori_loop(0, NCHUNK // _DEG_WAVE, wave, 0)
    plsc.subcore_barrier()
    pltpu.sync_copy(acc.at[pl.ds(r0, ROWS_TILE)],
                    out_hbm.at[pl.ds(cid * N_PAD + r0, ROWS_TILE)])


_deg_kernel = functools.partial(
    pl.kernel,
    out_type=jax.ShapeDtypeStruct((NC * N_PAD,), jnp.float32),
    mesh=_MESH,
    scratch_types=[
        pltpu.VMEM((NCHUNK, B), jnp.int32),
        pltpu.VMEM((NCHUNK, B), jnp.float32),
        pltpu.VMEM_SHARED((N_PAD,), jnp.float32),
        pltpu.SemaphoreType.DMA,
    ],
    compiler_params=_SC_PARAMS,
)(_deg_body)


def _make_edge_pass(D):
    def body(y_hbm, row3_hbm, col3_hbm, ew3_hbm, zero_hbm, out_hbm,
             row_all, col_all, ew_all, bf0, bf1, sbuf, acc, g0, g1, s0):
        cid = lax.axis_index("c")
        sid = lax.axis_index("s")
        w = cid * NS + sid
        r0 = sid * ROWS_TILE
        pltpu.sync_copy(zero_hbm.at[pl.ds(r0, ROWS_TILE)],
                        acc.at[pl.ds(r0, ROWS_TILE)])
        plsc.subcore_barrier()

        def start_gather(j, buf, sem):
            pltpu.async_copy(y_hbm.at[row_all.at[j]], buf, sem)

        def wait_gather(buf, sem):
            pltpu.make_async_copy(y_hbm.at[row_all.at[0]], buf, sem).wait()

        def start_scat(j):
            pltpu.async_copy(sbuf, acc.at[col_all.at[j]], s0, add=True)

        def wait_scat():
            pltpu.make_async_copy(sbuf, acc.at[col_all.at[0]], s0).wait()

        UNROLL = 4
        MASK = jnp.int32(-65536)

        def scale(bf, j):
            jvec = jnp.full((LANES,), j, jnp.int32)

            def one(i, evec):
                e0 = i * UNROLL
                svs = [plsc.load_gather(ew_all, [jvec, evec + u])
                       for u in range(UNROLL)]
                for u in range(UNROLL):
                    e = e0 + u
                    for g in range(D // 32):
                        v = bf[e, pl.ds(32 * g, 32)]
                        wi = plsc.bitcast(v, jnp.int32)
                        ev = plsc.bitcast(wi << 16, jnp.float32)
                        od = plsc.bitcast(wi & MASK, jnp.float32)
                        sbuf[e, pl.ds(32 * g, LANES)] = ev * svs[u]
                        sbuf[e, pl.ds(32 * g + LANES, LANES)] = od * svs[u]
                return evec + UNROLL

            lax.fori_loop(0, B // UNROLL, one,
                          jnp.zeros((LANES,), jnp.int32))

        nloop = CHUNK_P // 2

        def phase(p, carry):
            base = p * CHUNK_P
            d1 = pltpu.async_copy(row3_hbm.at[w, pl.ds(base, CHUNK_P)],
                                  row_all, g0)
            d2 = pltpu.async_copy(col3_hbm.at[w, pl.ds(base, CHUNK_P)],
                                  col_all, g0)
            d3 = pltpu.async_copy(ew3_hbm.at[w, pl.ds(base, CHUNK_P)],
                                  ew_all, g0)
            d1.wait()
            d2.wait()
            d3.wait()

            start_gather(0, bf0, g0)

            def body2(jj, carry2):
                j0 = jj * 2
                j1 = j0 + 1
                wait_gather(bf0, g0)

                @pl.when(jj > 0)
                def _():
                    wait_scat()

                start_gather(j1, bf1, g1)
                scale(bf0, j0)
                start_scat(j0)

                wait_gather(bf1, g1)
                wait_scat()

                @pl.when(jj < nloop - 1)
                def _():
                    start_gather(j0 + 2, bf0, g0)

                scale(bf1, j1)
                start_scat(j1)
                return carry2

            lax.fori_loop(0, nloop, body2, 0)
            wait_scat()
            return carry

        lax.fori_loop(0, NPHASE, phase, 0)
        plsc.subcore_barrier()
        pltpu.sync_copy(acc.at[pl.ds(r0, ROWS_TILE)],
                        out_hbm.at[pl.ds(cid * N_PAD + r0, ROWS_TILE)])

    return pl.kernel(
        body,
        out_type=jax.ShapeDtypeStruct((NC * N_PAD, D), jnp.float32),
        mesh=_MESH,
        scratch_types=[
            pltpu.VMEM((CHUNK_P, B), jnp.int32),
            pltpu.VMEM((CHUNK_P, B), jnp.int32),
            pltpu.VMEM((CHUNK_P, B), jnp.float32),
            pltpu.VMEM((B, D), jnp.bfloat16),
            pltpu.VMEM((B, D), jnp.bfloat16),
            pltpu.VMEM((B, D), jnp.float32),
            pltpu.VMEM_SHARED((N_PAD, D), jnp.float32),
            pltpu.SemaphoreType.DMA,
            pltpu.SemaphoreType.DMA,
            pltpu.SemaphoreType.DMA,
        ],
        compiler_params=_SC_PARAMS,
    )


_edge_pass_128 = _make_edge_pass(D_H)
_edge_pass_64 = _make_edge_pass(D_OUT)


def kernel(x, edge_index, edge_feature, W1, b1, W2, b2):
    row = edge_index[0].astype(jnp.int32)
    col = edge_index[1].astype(jnp.int32)
    pad = E_PAD - E
    row_p = jnp.concatenate([row, jnp.zeros((pad,), jnp.int32)]
                            ).reshape(NW, NCHUNK, B)
    col_p = jnp.concatenate([col, jnp.zeros((pad,), jnp.int32)]
                            ).reshape(NW, NCHUNK, B)

    ew2d = _tc1_call(edge_feature)
    ew_p = jnp.concatenate([ew2d[:, 0], jnp.zeros((pad,), jnp.float32)]
                           ).reshape(NW, NCHUNK, B)

    zeros1 = jnp.zeros((N_PAD,), jnp.float32)
    degp = _deg_kernel(col_p, ew_p, zeros1).reshape(NC, N_PAD)

    dinv, y1 = _tc_call(_tc2_body, [(N, 1), (N, D_H)], degp, x, W1)

    zeros_h = jnp.zeros((N_PAD, D_H), jnp.float32)
    y1bf = y1.astype(jnp.bfloat16)[:, _Q128]
    p = _edge_pass_128(y1bf, row_p, col_p, ew_p,
                       zeros_h).reshape(NC, N_PAD, D_H)

    y2, = _tc_call(_tc3_body, [(N, D_OUT)],
                   p[0, :N], p[1, :N], y1, dinv, b1.reshape(1, D_H), W2)

    zeros_o = jnp.zeros((N_PAD, D_OUT), jnp.float32)
    y2bf = y2.astype(jnp.bfloat16)[:, _Q64]
    q = _edge_pass_64(y2bf, row_p, col_p, ew_p,
                      zeros_o).reshape(NC, N_PAD, D_OUT)

    out, = _tc_call(_tc4_body, [(N, D_OUT)],
                    q[0, :N], q[1, :N], y2, dinv, b2.reshape(1, D_OUT))
    return (out, out)


# speedup vs baseline: 10.8142x; 1.0032x over previous
import functools

import numpy as np

import jax
import jax.numpy as jnp
from jax import lax
from jax.experimental import pallas as pl
from jax.experimental.pallas import tpu as pltpu
from jax.experimental.pallas import tpu_sc as plsc

N = 10000
E = 320000
D_IN = 128
D_H = 128
D_OUT = 64
D_EDGE_ = 16

NC = 2
NS = 16
LANES = 16

B = 128
NCHUNK = 80
NE_TILE = B * NCHUNK
NW = NC * NS
E_PAD = NE_TILE * NW
N_PAD = 10240
ROWS_TILE = N_PAD // NS
NPHASE = 4
CHUNK_P = NCHUNK // NPHASE


def _bf16_col_order(D):
    perm = np.empty((D,), np.int64)
    for g in range(D // 32):
        for k in range(16):
            perm[32 * g + k] = 32 * g + 2 * k
            perm[32 * g + 16 + k] = 32 * g + 2 * k + 1
    return np.argsort(perm)


_Q128 = _bf16_col_order(D_H)
_Q64 = _bf16_col_order(D_OUT)

_MESH = plsc.VectorSubcoreMesh(
    core_axis_name="c", subcore_axis_name="s", num_cores=NC, num_subcores=NS)
_SC_PARAMS = pltpu.CompilerParams(needs_layout_passes=False,
                                  use_tc_tiling_on_sc=False)


def _tc1_body(ef_ref, ew_ref):
    ew_ref[...] = jnp.mean(ef_ref[...], axis=1, keepdims=True)


_EW_GRID = 40
_EW_BLK = E // _EW_GRID


def _tc1_call(edge_feature):
    return pl.pallas_call(
        _tc1_body,
        grid=(_EW_GRID,),
        in_specs=[pl.BlockSpec((_EW_BLK, D_EDGE_), lambda i: (i, 0))],
        out_specs=pl.BlockSpec((_EW_BLK, 1), lambda i: (i, 0)),
        out_shape=jax.ShapeDtypeStruct((E, 1), jnp.float32),
    )(edge_feature)


def _tc2_body(degp_ref, x_ref, w1_ref, dinv_ref, y_ref):
    d = degp_ref[...]
    deg = d[0, :N] + d[1, :N] + 1.0
    dinv = jnp.where(deg > 0, lax.rsqrt(deg), 0.0)[:, None]
    dinv_ref[...] = dinv
    y_ref[...] = dinv * jnp.dot(x_ref[...], w1_ref[...],
                                preferred_element_type=jnp.float32)


def _tc3_body(p0_ref, p1_ref, y1_ref, dinv_ref, b1_ref, w2_ref, y2_ref):
    dinv = dinv_ref[...]
    h = jnp.maximum(dinv * (p0_ref[...] + p1_ref[...] + y1_ref[...])
                    + b1_ref[...], 0.0)
    y2_ref[...] = dinv * jnp.dot(h, w2_ref[...],
                                 preferred_element_type=jnp.float32)


def _tc4_body(q0_ref, q1_ref, y2_ref, dinv_ref, b2_ref, out_ref):
    out_ref[...] = (dinv_ref[...] * (q0_ref[...] + q1_ref[...] + y2_ref[...])
                    + b2_ref[...])


def _tc_call(body, out_shapes, *args):
    return pl.pallas_call(
        body,
        out_shape=[jax.ShapeDtypeStruct(s, jnp.float32) for s in out_shapes],
    )(*args)


_DEG_WAVE = 16


def _deg_body(col3_hbm, ew3_hbm, zero_hbm, out_hbm, idx_all, val_all, acc, sem):
    cid = lax.axis_index("c")
    sid = lax.axis_index("s")
    w = cid * NS + sid
    r0 = sid * ROWS_TILE
    pltpu.sync_copy(zero_hbm.at[pl.ds(r0, ROWS_TILE)],
                    acc.at[pl.ds(r0, ROWS_TILE)])
    pltpu.sync_copy(col3_hbm.at[w], idx_all)
    pltpu.sync_copy(ew3_hbm.at[w], val_all)
    plsc.subcore_barrier()

    def wave(t, carry):
        for k in range(_DEG_WAVE):
            j = t * _DEG_WAVE + k
            pltpu.async_copy(val_all.at[j], acc.at[idx_all.at[j]], sem,
                             add=True)
        for k in range(_DEG_WAVE):
            pltpu.make_async_copy(val_all.at[0], acc.at[idx_all.at[0]],
                                  sem).wait()
        return carry

    lax.fori_loop(0, NCHUNK // _DEG_WAVE, wave, 0)
    plsc.subcore_barrier()
    pltpu.sync_copy(acc.at[pl.ds(r0, ROWS_TILE)],
                    out_hbm.at[pl.ds(cid * N_PAD + r0, ROWS_TILE)])


_deg_kernel = functools.partial(
    pl.kernel,
    out_type=jax.ShapeDtypeStruct((NC * N_PAD,), jnp.float32),
    mesh=_MESH,
    scratch_types=[
        pltpu.VMEM((NCHUNK, B), jnp.int32),
        pltpu.VMEM((NCHUNK, B), jnp.float32),
        pltpu.VMEM_SHARED((N_PAD,), jnp.float32),
        pltpu.SemaphoreType.DMA,
    ],
    compiler_params=_SC_PARAMS,
)(_deg_body)


def _make_edge_pass(D):
    def body(y_hbm, row3_hbm, col3_hbm, ew3_hbm, out_hbm,
             row_all, col_all, ew_all, bf0, bf1, sbuf, acc, g0, g1, s0):
        cid = lax.axis_index("c")
        sid = lax.axis_index("s")
        w = cid * NS + sid
        r0 = sid * ROWS_TILE

        def zero_sbuf(e, carry):
            for g in range(D // LANES):
                sbuf[e, pl.ds(g * LANES, LANES)] = jnp.zeros((LANES,),
                                                             jnp.float32)
            return carry

        lax.fori_loop(0, B, zero_sbuf, 0)
        for t in range(ROWS_TILE // B):
            pltpu.sync_copy(sbuf, acc.at[pl.ds(r0 + t * B, B)])
        plsc.subcore_barrier()

        def start_gather(j, buf, sem):
            pltpu.async_copy(y_hbm.at[row_all.at[j]], buf, sem)

        def wait_gather(buf, sem):
            pltpu.make_async_copy(y_hbm.at[row_all.at[0]], buf, sem).wait()

        def start_scat(j):
            pltpu.async_copy(sbuf, acc.at[col_all.at[j]], s0, add=True)

        def wait_scat():
            pltpu.make_async_copy(sbuf, acc.at[col_all.at[0]], s0).wait()

        UNROLL = 8
        MASK = jnp.int32(-65536)

        def scale(bf, j):
            jvec = jnp.full((LANES,), j, jnp.int32)

            def one(i, evec):
                e0 = i * UNROLL
                svs = [plsc.load_gather(ew_all, [jvec, evec + u])
                       for u in range(UNROLL)]
                for u in range(UNROLL):
                    e = e0 + u
                    for g in range(D // 32):
                        v = bf[e, pl.ds(32 * g, 32)]
                        wi = plsc.bitcast(v, jnp.int32)
                        ev = plsc.bitcast(wi << 16, jnp.float32)
                        od = plsc.bitcast(wi & MASK, jnp.float32)
                        sbuf[e, pl.ds(32 * g, LANES)] = ev * svs[u]
                        sbuf[e, pl.ds(32 * g + LANES, LANES)] = od * svs[u]
                return evec + UNROLL

            lax.fori_loop(0, B // UNROLL, one,
                          jnp.zeros((LANES,), jnp.int32))

        nloop = CHUNK_P // 2

        def phase(p, carry):
            base = p * CHUNK_P
            d1 = pltpu.async_copy(row3_hbm.at[w, pl.ds(base, CHUNK_P)],
                                  row_all, g0)
            d2 = pltpu.async_copy(col3_hbm.at[w, pl.ds(base, CHUNK_P)],
                                  col_all, g0)
            d3 = pltpu.async_copy(ew3_hbm.at[w, pl.ds(base, CHUNK_P)],
                                  ew_all, g0)
            d1.wait()
            d2.wait()
            d3.wait()

            start_gather(0, bf0, g0)

            def body2(jj, carry2):
                j0 = jj * 2
                j1 = j0 + 1
                wait_gather(bf0, g0)

                @pl.when(jj > 0)
                def _():
                    wait_scat()

                start_gather(j1, bf1, g1)
                scale(bf0, j0)
                start_scat(j0)

                wait_gather(bf1, g1)
                wait_scat()

                @pl.when(jj < nloop - 1)
                def _():
                    start_gather(j0 + 2, bf0, g0)

                scale(bf1, j1)
                start_scat(j1)
                return carry2

            lax.fori_loop(0, nloop, body2, 0)
            wait_scat()
            return carry

        lax.fori_loop(0, NPHASE, phase, 0)
        plsc.subcore_barrier()
        pltpu.sync_copy(acc.at[pl.ds(r0, ROWS_TILE)],
                        out_hbm.at[pl.ds(cid * N_PAD + r0, ROWS_TILE)])

    return pl.kernel(
        body,
        out_type=jax.ShapeDtypeStruct((NC * N_PAD, D), jnp.float32),
        mesh=_MESH,
        scratch_types=[
            pltpu.VMEM((CHUNK_P, B), jnp.int32),
            pltpu.VMEM((CHUNK_P, B), jnp.int32),
            pltpu.VMEM((CHUNK_P, B), jnp.float32),
            pltpu.VMEM((B, D), jnp.bfloat16),
            pltpu.VMEM((B, D), jnp.bfloat16),
            pltpu.VMEM((B, D), jnp.float32),
            pltpu.VMEM_SHARED((N_PAD, D), jnp.float32),
            pltpu.SemaphoreType.DMA,
            pltpu.SemaphoreType.DMA,
            pltpu.SemaphoreType.DMA,
        ],
        compiler_params=_SC_PARAMS,
    )


_edge_pass_128 = _make_edge_pass(D_H)
_edge_pass_64 = _make_edge_pass(D_OUT)


def kernel(x, edge_index, edge_feature, W1, b1, W2, b2):
    row = edge_index[0].astype(jnp.int32)
    col = edge_index[1].astype(jnp.int32)
    pad = E_PAD - E
    row_p = jnp.concatenate([row, jnp.zeros((pad,), jnp.int32)]
                            ).reshape(NW, NCHUNK, B)
    col_p = jnp.concatenate([col, jnp.zeros((pad,), jnp.int32)]
                            ).reshape(NW, NCHUNK, B)

    ew2d = _tc1_call(edge_feature)
    ew_p = jnp.concatenate([ew2d[:, 0], jnp.zeros((pad,), jnp.float32)]
                           ).reshape(NW, NCHUNK, B)

    zeros1 = jnp.zeros((N_PAD,), jnp.float32)
    degp = _deg_kernel(col_p, ew_p, zeros1).reshape(NC, N_PAD)

    dinv, y1 = _tc_call(_tc2_body, [(N, 1), (N, D_H)], degp, x, W1)

    y1bf = y1.astype(jnp.bfloat16)[:, _Q128]
    p = _edge_pass_128(y1bf, row_p, col_p, ew_p).reshape(NC, N_PAD, D_H)

    y2, = _tc_call(_tc3_body, [(N, D_OUT)],
                   p[0, :N], p[1, :N], y1, dinv, b1.reshape(1, D_H), W2)

    y2bf = y2.astype(jnp.bfloat16)[:, _Q64]
    q = _edge_pass_64(y2bf, row_p, col_p, ew_p).reshape(NC, N_PAD, D_OUT)

    out, = _tc_call(_tc4_body, [(N, D_OUT)],
                    q[0, :N], q[1, :N], y2, dinv, b2.reshape(1, D_OUT))
    return (out, out)


# speedup vs baseline: 10.8475x; 1.0031x over previous
import functools

import numpy as np

import jax
import jax.numpy as jnp
from jax import lax
from jax.experimental import pallas as pl
from jax.experimental.pallas import tpu as pltpu
from jax.experimental.pallas import tpu_sc as plsc

N = 10000
E = 320000
D_IN = 128
D_H = 128
D_OUT = 64
D_EDGE_ = 16

NC = 2
NS = 16
LANES = 16

B = 128
NCHUNK = 80
NE_TILE = B * NCHUNK
NW = NC * NS
E_PAD = NE_TILE * NW
N_PAD = 10240
ROWS_TILE = N_PAD // NS


def _bf16_col_order(D):
    perm = np.empty((D,), np.int64)
    for g in range(D // 32):
        for k in range(16):
            perm[32 * g + k] = 32 * g + 2 * k
            perm[32 * g + 16 + k] = 32 * g + 2 * k + 1
    return np.argsort(perm)


_Q128 = _bf16_col_order(D_H)
_Q64 = _bf16_col_order(D_OUT)

_MESH = plsc.VectorSubcoreMesh(
    core_axis_name="c", subcore_axis_name="s", num_cores=NC, num_subcores=NS)
_SC_PARAMS = pltpu.CompilerParams(needs_layout_passes=False,
                                  use_tc_tiling_on_sc=False)


def _tc1_body(ef_ref, ew_ref):
    ew_ref[...] = jnp.mean(ef_ref[...], axis=1, keepdims=True)


_EW_GRID = 40
_EW_BLK = E // _EW_GRID


def _tc1_call(edge_feature):
    return pl.pallas_call(
        _tc1_body,
        grid=(_EW_GRID,),
        in_specs=[pl.BlockSpec((_EW_BLK, D_EDGE_), lambda i: (i, 0))],
        out_specs=pl.BlockSpec((_EW_BLK, 1), lambda i: (i, 0)),
        out_shape=jax.ShapeDtypeStruct((E, 1), jnp.float32),
    )(edge_feature)


def _tc2_body(degp_ref, x_ref, w1_ref, dinv_ref, y_ref):
    d = degp_ref[...]
    deg = d[0, :N] + d[1, :N] + 1.0
    dinv = jnp.where(deg > 0, lax.rsqrt(deg), 0.0)[:, None]
    dinv_ref[...] = dinv
    y_ref[...] = dinv * jnp.dot(x_ref[...], w1_ref[...],
                                preferred_element_type=jnp.float32)


def _tc3_body(p0_ref, p1_ref, y1_ref, dinv_ref, b1_ref, w2_ref, y2_ref):
    dinv = dinv_ref[...]
    h = jnp.maximum(dinv * (p0_ref[...] + p1_ref[...] + y1_ref[...])
                    + b1_ref[...], 0.0)
    y2_ref[...] = dinv * jnp.dot(h, w2_ref[...],
                                 preferred_element_type=jnp.float32)


def _tc4_body(q0_ref, q1_ref, y2_ref, dinv_ref, b2_ref, out_ref):
    out_ref[...] = (dinv_ref[...] * (q0_ref[...] + q1_ref[...] + y2_ref[...])
                    + b2_ref[...])


def _tc_call(body, out_shapes, *args):
    return pl.pallas_call(
        body,
        out_shape=[jax.ShapeDtypeStruct(s, jnp.float32) for s in out_shapes],
    )(*args)


_DEG_WAVE = 16


def _deg_body(col3_hbm, ew3_hbm, zero_hbm, out_hbm, idx_all, val_all, acc, sem):
    cid = lax.axis_index("c")
    sid = lax.axis_index("s")
    w = cid * NS + sid
    r0 = sid * ROWS_TILE
    pltpu.sync_copy(zero_hbm.at[pl.ds(r0, ROWS_TILE)],
                    acc.at[pl.ds(r0, ROWS_TILE)])
    pltpu.sync_copy(col3_hbm.at[w], idx_all)
    pltpu.sync_copy(ew3_hbm.at[w], val_all)
    plsc.subcore_barrier()

    def wave(t, carry):
        for k in range(_DEG_WAVE):
            j = t * _DEG_WAVE + k
            pltpu.async_copy(val_all.at[j], acc.at[idx_all.at[j]], sem,
                             add=True)
        for k in range(_DEG_WAVE):
            pltpu.make_async_copy(val_all.at[0], acc.at[idx_all.at[0]],
                                  sem).wait()
        return carry

    lax.fori_loop(0, NCHUNK // _DEG_WAVE, wave, 0)
    plsc.subcore_barrier()
    pltpu.sync_copy(acc.at[pl.ds(r0, ROWS_TILE)],
                    out_hbm.at[pl.ds(cid * N_PAD + r0, ROWS_TILE)])


_deg_kernel = functools.partial(
    pl.kernel,
    out_type=jax.ShapeDtypeStruct((NC * N_PAD,), jnp.float32),
    mesh=_MESH,
    scratch_types=[
        pltpu.VMEM((NCHUNK, B), jnp.int32),
        pltpu.VMEM((NCHUNK, B), jnp.float32),
        pltpu.VMEM_SHARED((N_PAD,), jnp.float32),
        pltpu.SemaphoreType.DMA,
    ],
    compiler_params=_SC_PARAMS,
)(_deg_body)


def _make_edge_pass(D):
    NPHASE = 4 if D == D_H else 1
    CHUNK_P = NCHUNK // NPHASE

    def body(y_hbm, row3_hbm, col3_hbm, ew3_hbm, out_hbm,
             row_all, col_all, ew_all, bf0, bf1, sbuf, acc, g0, g1, s0):
        cid = lax.axis_index("c")
        sid = lax.axis_index("s")
        w = cid * NS + sid
        r0 = sid * ROWS_TILE

        def zero_sbuf(e, carry):
            for g in range(D // LANES):
                sbuf[e, pl.ds(g * LANES, LANES)] = jnp.zeros((LANES,),
                                                             jnp.float32)
            return carry

        lax.fori_loop(0, B, zero_sbuf, 0)
        for t in range(ROWS_TILE // B):
            pltpu.sync_copy(sbuf, acc.at[pl.ds(r0 + t * B, B)])
        plsc.subcore_barrier()

        def start_gather(j, buf, sem):
            pltpu.async_copy(y_hbm.at[row_all.at[j]], buf, sem)

        def wait_gather(buf, sem):
            pltpu.make_async_copy(y_hbm.at[row_all.at[0]], buf, sem).wait()

        def start_scat(j):
            pltpu.async_copy(sbuf, acc.at[col_all.at[j]], s0, add=True)

        def wait_scat():
            pltpu.make_async_copy(sbuf, acc.at[col_all.at[0]], s0).wait()

        UNROLL = 8
        MASK = jnp.int32(-65536)

        def scale(bf, j):
            jvec = jnp.full((LANES,), j, jnp.int32)

            def one(i, evec):
                e0 = i * UNROLL
                svs = [plsc.load_gather(ew_all, [jvec, evec + u])
                       for u in range(UNROLL)]
                for u in range(UNROLL):
                    e = e0 + u
                    for g in range(D // 32):
                        v = bf[e, pl.ds(32 * g, 32)]
                        wi = plsc.bitcast(v, jnp.int32)
                        ev = plsc.bitcast(wi << 16, jnp.float32)
                        od = plsc.bitcast(wi & MASK, jnp.float32)
                        sbuf[e, pl.ds(32 * g, LANES)] = ev * svs[u]
                        sbuf[e, pl.ds(32 * g + LANES, LANES)] = od * svs[u]
                return evec + UNROLL

            lax.fori_loop(0, B // UNROLL, one,
                          jnp.zeros((LANES,), jnp.int32))

        nloop = CHUNK_P // 2

        def phase(p, carry):
            base = p * CHUNK_P
            d1 = pltpu.async_copy(row3_hbm.at[w, pl.ds(base, CHUNK_P)],
                                  row_all, g0)
            d2 = pltpu.async_copy(col3_hbm.at[w, pl.ds(base, CHUNK_P)],
                                  col_all, g0)
            d3 = pltpu.async_copy(ew3_hbm.at[w, pl.ds(base, CHUNK_P)],
                                  ew_all, g0)
            d1.wait()
            d2.wait()
            d3.wait()

            start_gather(0, bf0, g0)

            def body2(jj, carry2):
                j0 = jj * 2
                j1 = j0 + 1
                wait_gather(bf0, g0)

                @pl.when(jj > 0)
                def _():
                    wait_scat()

                start_gather(j1, bf1, g1)
                scale(bf0, j0)
                start_scat(j0)

                wait_gather(bf1, g1)
                wait_scat()

                @pl.when(jj < nloop - 1)
                def _():
                    start_gather(j0 + 2, bf0, g0)

                scale(bf1, j1)
                start_scat(j1)
                return carry2

            lax.fori_loop(0, nloop, body2, 0)
            wait_scat()
            return carry

        lax.fori_loop(0, NPHASE, phase, 0)
        plsc.subcore_barrier()
        pltpu.sync_copy(acc.at[pl.ds(r0, ROWS_TILE)],
                        out_hbm.at[pl.ds(cid * N_PAD + r0, ROWS_TILE)])

    return pl.kernel(
        body,
        out_type=jax.ShapeDtypeStruct((NC * N_PAD, D), jnp.float32),
        mesh=_MESH,
        scratch_types=[
            pltpu.VMEM((CHUNK_P, B), jnp.int32),
            pltpu.VMEM((CHUNK_P, B), jnp.int32),
            pltpu.VMEM((CHUNK_P, B), jnp.float32),
            pltpu.VMEM((B, D), jnp.bfloat16),
            pltpu.VMEM((B, D), jnp.bfloat16),
            pltpu.VMEM((B, D), jnp.float32),
            pltpu.VMEM_SHARED((N_PAD, D), jnp.float32),
            pltpu.SemaphoreType.DMA,
            pltpu.SemaphoreType.DMA,
            pltpu.SemaphoreType.DMA,
        ],
        compiler_params=_SC_PARAMS,
    )


_edge_pass_128 = _make_edge_pass(D_H)
_edge_pass_64 = _make_edge_pass(D_OUT)


def kernel(x, edge_index, edge_feature, W1, b1, W2, b2):
    row = edge_index[0].astype(jnp.int32)
    col = edge_index[1].astype(jnp.int32)
    pad = E_PAD - E
    row_p = jnp.concatenate([row, jnp.zeros((pad,), jnp.int32)]
                            ).reshape(NW, NCHUNK, B)
    col_p = jnp.concatenate([col, jnp.zeros((pad,), jnp.int32)]
                            ).reshape(NW, NCHUNK, B)

    ew2d = _tc1_call(edge_feature)
    ew_p = jnp.concatenate([ew2d[:, 0], jnp.zeros((pad,), jnp.float32)]
                           ).reshape(NW, NCHUNK, B)

    zeros1 = jnp.zeros((N_PAD,), jnp.float32)
    degp = _deg_kernel(col_p, ew_p, zeros1).reshape(NC, N_PAD)

    dinv, y1 = _tc_call(_tc2_body, [(N, 1), (N, D_H)], degp, x, W1)

    y1bf = y1.astype(jnp.bfloat16)[:, _Q128]
    p = _edge_pass_128(y1bf, row_p, col_p, ew_p).reshape(NC, N_PAD, D_H)

    y2, = _tc_call(_tc3_body, [(N, D_OUT)],
                   p[0, :N], p[1, :N], y1, dinv, b1.reshape(1, D_H), W2)

    y2bf = y2.astype(jnp.bfloat16)[:, _Q64]
    q = _edge_pass_64(y2bf, row_p, col_p, ew_p).reshape(NC, N_PAD, D_OUT)

    out, = _tc_call(_tc4_body, [(N, D_OUT)],
                    q[0, :N], q[1, :N], y2, dinv, b2.reshape(1, D_OUT))
    return (out, out)
